# baseline (speedup 1.0000x reference)
import functools

import jax
import jax.numpy as jnp
from jax import lax
from jax.experimental import pallas as pl
from jax.experimental.pallas import tpu as pltpu
from jax.experimental.pallas import tpu_sc as plsc

N_NODES = 10000
N_EDGES = 160000
NODE_DIM = 128
EDGE_DIM = 16
S = 48
NUM_CONV = 3
B = 64

NW = 32
CHUNK = 128
E_PAD = 163840
EDGES_PER_TILE = E_PAD // NW
N_CHUNKS = EDGES_PER_TILE // CHUNK
AW = 128
OW = 128


def _in_linear_body(x_ref, w_ref, b_ref, o_ref):
    o = jax.nn.relu(
        jnp.dot(x_ref[...], w_ref[...], preferred_element_type=jnp.float32)
        + b_ref[...])
    o_ref[...] = jnp.concatenate(
        [o, jnp.zeros((N_NODES, OW - S), jnp.float32)], axis=1)


def _in_linear(x, w, b):
    return pl.pallas_call(
        _in_linear_body,
        out_shape=jax.ShapeDtypeStruct((N_NODES, OW), jnp.float32),
    )(x, w, b)


def _eh_body(a_ref, w_ref, b_ref, o_ref):
    o_ref[...] = jax.nn.relu(
        jnp.dot(a_ref[...], w_ref[...], preferred_element_type=jnp.float32)
        + b_ref[...])


def _eh(edge_attr_pad, w, b):
    blk = 8192
    grid = E_PAD // blk
    return pl.pallas_call(
        _eh_body,
        grid=(grid,),
        in_specs=[
            pl.BlockSpec((blk, EDGE_DIM), lambda i: (i, 0)),
            pl.BlockSpec((EDGE_DIM, S), lambda i: (0, 0)),
            pl.BlockSpec((1, S), lambda i: (0, 0)),
        ],
        out_specs=pl.BlockSpec((blk, S), lambda i: (i, 0)),
        out_shape=jax.ShapeDtypeStruct((E_PAD, S), jnp.float32),
    )(edge_attr_pad, w, b)


_MSG_EB = 1280


def _msg_body(osrc_ref, eh_ref, g_ref, bm_ref, o_ref):
    osrc = osrc_ref[...][:, :S]
    ehb = eh_ref[...]
    parts = [osrc[:, i:i + 1] * ehb for i in range(S)]
    r = jnp.concatenate(parts, axis=1)
    msg = jnp.dot(r, g_ref[...], preferred_element_type=jnp.float32)
    msg = msg + jnp.dot(osrc, bm_ref[...], preferred_element_type=jnp.float32)
    eidx = (pl.program_id(0) * _MSG_EB
            + lax.broadcasted_iota(jnp.int32, (_MSG_EB, 1), 0))
    valid = (eidx < N_EDGES).astype(jnp.float32)
    o_ref[...] = jnp.concatenate(
        [msg * valid, valid, jnp.zeros((_MSG_EB, AW - S - 1), jnp.float32)],
        axis=1)


def _msg(out_src, eh, gflat, bmat):
    grid = E_PAD // _MSG_EB
    return pl.pallas_call(
        _msg_body,
        grid=(grid,),
        in_specs=[
            pl.BlockSpec((_MSG_EB, OW), lambda i: (i, 0)),
            pl.BlockSpec((_MSG_EB, S), lambda i: (i, 0)),
            pl.BlockSpec((S * S, S), lambda i: (0, 0)),
            pl.BlockSpec((S, S), lambda i: (0, 0)),
        ],
        out_specs=pl.BlockSpec((_MSG_EB, AW), lambda i: (i, 0)),
        out_shape=jax.ShapeDtypeStruct((E_PAD, AW), jnp.float32),
    )(out_src, eh, gflat, bmat)


def _gru_body(p_ref, h_ref, wih_ref, whh_ref, bih_ref, bhh_ref, cb_ref, o_ref):
    p = p_ref[0] + p_ref[1]
    agg = p[:, :S]
    deg = jnp.maximum(p[:, S:S + 1], 1.0)
    m = jax.nn.relu(agg / deg + cb_ref[...])
    h = h_ref[...][:, :S]
    gi = jnp.dot(m, wih_ref[...], preferred_element_type=jnp.float32) + bih_ref[...]
    gh = jnp.dot(h, whh_ref[...], preferred_element_type=jnp.float32) + bhh_ref[...]
    r = jax.nn.sigmoid(gi[:, :S] + gh[:, :S])
    z = jax.nn.sigmoid(gi[:, S:2 * S] + gh[:, S:2 * S])
    n = jnp.tanh(gi[:, 2 * S:] + r * gh[:, 2 * S:])
    hn = (1.0 - z) * n + z * h
    o_ref[...] = jnp.concatenate(
        [hn, jnp.zeros((N_NODES, OW - S), jnp.float32)], axis=1)


def _gru(partials, h, wih, whh, bih, bhh, cb):
    return pl.pallas_call(
        _gru_body,
        out_shape=jax.ShapeDtypeStruct((N_NODES, OW), jnp.float32),
    )(partials, h, wih, whh, bih, bhh, cb)


def _set2set_body(out_ref, batch_ref, wih_ref, whh_ref, bih_ref, bhh_ref,
                  wo1_ref, bo1_ref, wo2_ref, bo2_ref, y_ref):
    out = out_ref[...][:, :S]
    bm = batch_ref[...]
    onehot = (bm == lax.broadcasted_iota(jnp.int32, (1, B), 1)
              ).astype(jnp.float32)
    q_star = jnp.zeros((B, 2 * S), jnp.float32)
    hs = jnp.zeros((B, S), jnp.float32)
    cs = jnp.zeros((B, S), jnp.float32)
    for _ in range(3):
        g = (jnp.dot(q_star, wih_ref[...], preferred_element_type=jnp.float32)
             + bih_ref[...]
             + jnp.dot(hs, whh_ref[...], preferred_element_type=jnp.float32)
             + bhh_ref[...])
        i = jax.nn.sigmoid(g[:, :S])
        f = jax.nn.sigmoid(g[:, S:2 * S])
        gc = jnp.tanh(g[:, 2 * S:3 * S])
        o = jax.nn.sigmoid(g[:, 3 * S:])
        cs = f * cs + i * gc
        hs = o * jnp.tanh(cs)
        q = hs
        qb = jnp.dot(onehot, q, preferred_element_type=jnp.float32)
        e = jnp.sum(out * qb, axis=-1, keepdims=True)
        emasked = jnp.where(onehot > 0.0, e, -jnp.inf)
        emax = jnp.max(emasked, axis=0, keepdims=True)
        emax_b = jnp.dot(onehot, emax.T, preferred_element_type=jnp.float32)
        ez = jnp.exp(e - emax_b)
        esum = jnp.dot(ez.T, onehot, preferred_element_type=jnp.float32)
        esum_b = jnp.dot(onehot, esum.T, preferred_element_type=jnp.float32)
        a = ez / esum_b
        rvec = jnp.dot(onehot.T, a * out, preferred_element_type=jnp.float32)
        q_star = jnp.concatenate([q, rvec], axis=1)
    hidden = jax.nn.relu(
        jnp.dot(q_star, wo1_ref[...], preferred_element_type=jnp.float32)
        + bo1_ref[...])
    y_ref[...] = (jnp.dot(hidden, wo2_ref[...],
                          preferred_element_type=jnp.float32) + bo2_ref[...])


def _set2set(out, batch2d, wih, whh, bih, bhh, wo1, bo1, wo2, bo2):
    return pl.pallas_call(
        _set2set_body,
        out_shape=jax.ShapeDtypeStruct((B, 1), jnp.float32),
    )(out, batch2d, wih, whh, bih, bhh, wo1, bo1, wo2, bo2)


@functools.lru_cache(maxsize=None)
def _make_sc_gather():
    mesh = plsc.VectorSubcoreMesh(core_axis_name="c", subcore_axis_name="s")

    @functools.partial(
        pl.kernel, mesh=mesh,
        out_type=jax.ShapeDtypeStruct((E_PAD, OW), jnp.float32),
        scratch_types=[
            pltpu.VMEM((CHUNK,), jnp.int32),
            pltpu.VMEM((CHUNK, OW), jnp.float32),
            pltpu.SemaphoreType.DMA,
        ],
    )
    def gather_k(table_hbm, idx_hbm, out_hbm, idx_v, rows_v, sem):
        wid = lax.axis_index("s") * 2 + lax.axis_index("c")
        base = wid * EDGES_PER_TILE

        def body(j, carry):
            off = base + j * CHUNK
            pltpu.sync_copy(idx_hbm.at[pl.ds(off, CHUNK)], idx_v)
            pltpu.async_copy(table_hbm.at[idx_v], rows_v, sem).wait()
            pltpu.sync_copy(rows_v, out_hbm.at[pl.ds(off, CHUNK)])
            return carry

        lax.fori_loop(0, N_CHUNKS, body, 0)

    return gather_k


def _sc_gather(table, idx):
    return _make_sc_gather()(table, idx)


@functools.lru_cache(maxsize=None)
def _make_sc_scatter():
    mesh = plsc.VectorSubcoreMesh(core_axis_name="c", subcore_axis_name="s")

    @functools.partial(
        pl.kernel, mesh=mesh,
        out_type=jax.ShapeDtypeStruct((2, N_NODES, AW), jnp.float32),
        scratch_types=[
            pltpu.VMEM((CHUNK,), jnp.int32),
            pltpu.VMEM((CHUNK, AW), jnp.float32),
            pltpu.VMEM_SHARED((N_NODES, AW), jnp.float32),
        ],
    )
    def scatter_k(msg_hbm, idx_hbm, zeros_hbm, out_hbm, idx_v, rows_v, acc_sh):
        cid = lax.axis_index("c")
        sid = lax.axis_index("s")
        wid = sid * 2 + cid

        @pl.when(sid == 0)
        def _():
            pltpu.sync_copy(zeros_hbm, acc_sh)

        plsc.subcore_barrier()
        base = wid * EDGES_PER_TILE

        def body(j, carry):
            off = base + j * CHUNK
            pltpu.sync_copy(idx_hbm.at[pl.ds(off, CHUNK)], idx_v)
            pltpu.sync_copy(msg_hbm.at[pl.ds(off, CHUNK)], rows_v)
            pltpu.sync_copy(rows_v, acc_sh.at[idx_v], add=True)
            return carry

        lax.fori_loop(0, N_CHUNKS, body, 0)
        plsc.subcore_barrier()

        @pl.when(sid == 0)
        def _():
            pltpu.sync_copy(acc_sh, out_hbm.at[cid])

    return scatter_k


def _sc_scatter(msg, idx, zeros):
    return _make_sc_scatter()(msg, idx, zeros)


def kernel(x, edge_index, edge_attr, batch, W_in, b_in, W_e1, b_e1, W_e2,
           b_e2, conv_bias, gru_W_ih, gru_W_hh, gru_b_ih, gru_b_hh,
           lstm_W_ih, lstm_W_hh, lstm_b_ih, lstm_b_hh, W_o1, b_o1, W_o2,
           b_o2):
    w_in_t = W_in.T
    b_in2 = b_in.reshape(1, S)
    w_e1_t = W_e1.T
    b_e12 = b_e1.reshape(1, S)
    gflat = W_e2.reshape(S, S, S).transpose(0, 2, 1).reshape(S * S, S)
    bmat = b_e2.reshape(S, S)
    cb2 = conv_bias.reshape(1, S)
    wih_t = gru_W_ih.T
    whh_t = gru_W_hh.T
    bih2 = gru_b_ih.reshape(1, 3 * S)
    bhh2 = gru_b_hh.reshape(1, 3 * S)
    lwih_t = lstm_W_ih.T
    lwhh_t = lstm_W_hh.T
    lbih2 = lstm_b_ih.reshape(1, 4 * S)
    lbhh2 = lstm_b_hh.reshape(1, 4 * S)
    wo1_t = W_o1.T
    bo12 = b_o1.reshape(1, S)
    wo2_t = W_o2.T
    bo22 = b_o2.reshape(1, 1)

    pad_e = E_PAD - N_EDGES
    src = jnp.pad(edge_index[0], (0, pad_e))
    dst = jnp.pad(edge_index[1], (0, pad_e))
    ea_pad = jnp.pad(edge_attr, ((0, pad_e), (0, 0)))
    batch2d = batch.reshape(N_NODES, 1)
    zeros_acc = jnp.zeros((N_NODES, AW), jnp.float32)

    out = _in_linear(x, w_in_t, b_in2)
    eh = _eh(ea_pad, w_e1_t, b_e12)

    h = out
    for _ in range(NUM_CONV):
        out_src = _sc_gather(out, src)
        msg = _msg(out_src, eh, gflat, bmat)
        partials = _sc_scatter(msg, dst, zeros_acc)
        h = _gru(partials, h, wih_t, whh_t, bih2, bhh2, cb2)
        out = h

    y = _set2set(out, batch2d, lwih_t, lwhh_t, lbih2, lbhh2,
                 wo1_t, bo12, wo2_t, bo22)
    return y.reshape(-1)


# speedup vs baseline: 1.8057x; 1.8057x over previous
import functools

import jax
import jax.numpy as jnp
from jax import lax
from jax.experimental import pallas as pl
from jax.experimental.pallas import tpu as pltpu
from jax.experimental.pallas import tpu_sc as plsc

N_NODES = 10000
N_EDGES = 160000
NODE_DIM = 128
EDGE_DIM = 16
S = 48
NUM_CONV = 3
B = 64

NW = 32
CHUNK = 128
E_PAD = 163840
EDGES_PER_TILE = E_PAD // NW
N_CHUNKS = EDGES_PER_TILE // CHUNK
AW = 128
OW = 128
NPAD = 10240


def _in_linear_body(x_ref, w_ref, b_ref, o_ref):
    o = jax.nn.relu(
        jnp.dot(x_ref[...], w_ref[...], preferred_element_type=jnp.float32)
        + b_ref[...])
    o_ref[...] = jnp.concatenate(
        [o, jnp.zeros((N_NODES, OW - S), jnp.float32)], axis=1)


def _in_linear(x, w, b):
    return pl.pallas_call(
        _in_linear_body,
        out_shape=jax.ShapeDtypeStruct((N_NODES, OW), jnp.float32),
    )(x, w, b)


def _eh_body(a_ref, w_ref, b_ref, o_ref):
    o_ref[...] = jax.nn.relu(
        jnp.dot(a_ref[...], w_ref[...], preferred_element_type=jnp.float32)
        + b_ref[...])


def _eh(edge_attr_pad, w, b):
    blk = 8192
    grid = E_PAD // blk
    return pl.pallas_call(
        _eh_body,
        grid=(grid,),
        in_specs=[
            pl.BlockSpec((blk, EDGE_DIM), lambda i: (i, 0)),
            pl.BlockSpec((EDGE_DIM, S), lambda i: (0, 0)),
            pl.BlockSpec((1, S), lambda i: (0, 0)),
        ],
        out_specs=pl.BlockSpec((blk, S), lambda i: (i, 0)),
        out_shape=jax.ShapeDtypeStruct((E_PAD, S), jnp.float32),
    )(edge_attr_pad, w, b)


_MSG_EB = 1280


def _msg_body(osrc_ref, eh_ref, g_ref, bm_ref, rep_ref, tile_ref, o_ref):
    osrc = osrc_ref[...][:, :S]
    ehb = eh_ref[...]
    osrc_rep = jnp.dot(osrc, rep_ref[...], preferred_element_type=jnp.float32)
    eh_tile = jnp.dot(ehb, tile_ref[...], preferred_element_type=jnp.float32)
    r = osrc_rep * eh_tile
    msg = jnp.dot(r, g_ref[...], preferred_element_type=jnp.float32)
    msg = msg + jnp.dot(osrc, bm_ref[...], preferred_element_type=jnp.float32)
    eidx = (pl.program_id(0) * _MSG_EB
            + lax.broadcasted_iota(jnp.int32, (_MSG_EB, 1), 0))
    valid = (eidx < N_EDGES).astype(jnp.float32)
    o_ref[...] = jnp.concatenate(
        [msg * valid, valid, jnp.zeros((_MSG_EB, AW - S - 1), jnp.float32)],
        axis=1)


def _msg(out_src, eh, gflat, bmat, rep, tile):
    grid = E_PAD // _MSG_EB
    return pl.pallas_call(
        _msg_body,
        grid=(grid,),
        in_specs=[
            pl.BlockSpec((_MSG_EB, OW), lambda i: (i, 0)),
            pl.BlockSpec((_MSG_EB, S), lambda i: (i, 0)),
            pl.BlockSpec((S * S, S), lambda i: (0, 0)),
            pl.BlockSpec((S, S), lambda i: (0, 0)),
            pl.BlockSpec((S, S * S), lambda i: (0, 0)),
            pl.BlockSpec((S, S * S), lambda i: (0, 0)),
        ],
        out_specs=pl.BlockSpec((_MSG_EB, AW), lambda i: (i, 0)),
        out_shape=jax.ShapeDtypeStruct((E_PAD, AW), jnp.float32),
    )(out_src, eh, gflat, bmat, rep, tile)


def _gru_body(p_ref, h_ref, wih_ref, whh_ref, bih_ref, bhh_ref, cb_ref, o_ref):
    p = p_ref[0] + p_ref[1]
    agg = p[:, :S]
    deg = jnp.maximum(p[:, S:S + 1], 1.0)
    m = jax.nn.relu(agg / deg + cb_ref[...])
    h = h_ref[...][:, :S]
    gi = jnp.dot(m, wih_ref[...], preferred_element_type=jnp.float32) + bih_ref[...]
    gh = jnp.dot(h, whh_ref[...], preferred_element_type=jnp.float32) + bhh_ref[...]
    r = jax.nn.sigmoid(gi[:, :S] + gh[:, :S])
    z = jax.nn.sigmoid(gi[:, S:2 * S] + gh[:, S:2 * S])
    n = jnp.tanh(gi[:, 2 * S:] + r * gh[:, 2 * S:])
    hn = (1.0 - z) * n + z * h
    o_ref[...] = jnp.concatenate(
        [hn, jnp.zeros((N_NODES, OW - S), jnp.float32)], axis=1)


def _gru(partials, h, wih, whh, bih, bhh, cb):
    return pl.pallas_call(
        _gru_body,
        out_shape=jax.ShapeDtypeStruct((N_NODES, OW), jnp.float32),
    )(partials, h, wih, whh, bih, bhh, cb)


def _set2set_body(out_ref, batch_ref, wih_ref, whh_ref, bih_ref, bhh_ref,
                  wo1_ref, bo1_ref, wo2_ref, bo2_ref, y_ref):
    out = out_ref[...][:, :S]
    bm = batch_ref[...]
    onehot = (bm == lax.broadcasted_iota(jnp.int32, (1, B), 1)
              ).astype(jnp.float32)
    q_star = jnp.zeros((B, 2 * S), jnp.float32)
    hs = jnp.zeros((B, S), jnp.float32)
    cs = jnp.zeros((B, S), jnp.float32)
    for _ in range(3):
        g = (jnp.dot(q_star, wih_ref[...], preferred_element_type=jnp.float32)
             + bih_ref[...]
             + jnp.dot(hs, whh_ref[...], preferred_element_type=jnp.float32)
             + bhh_ref[...])
        i = jax.nn.sigmoid(g[:, :S])
        f = jax.nn.sigmoid(g[:, S:2 * S])
        gc = jnp.tanh(g[:, 2 * S:3 * S])
        o = jax.nn.sigmoid(g[:, 3 * S:])
        cs = f * cs + i * gc
        hs = o * jnp.tanh(cs)
        q = hs
        qb = jnp.dot(onehot, q, preferred_element_type=jnp.float32)
        e = jnp.sum(out * qb, axis=-1, keepdims=True)
        emasked = jnp.where(onehot > 0.0, e, -jnp.inf)
        emax = jnp.max(emasked, axis=0, keepdims=True)
        emax_b = jnp.dot(onehot, emax.T, preferred_element_type=jnp.float32)
        ez = jnp.exp(e - emax_b)
        esum = jnp.dot(ez.T, onehot, preferred_element_type=jnp.float32)
        esum_b = jnp.dot(onehot, esum.T, preferred_element_type=jnp.float32)
        a = ez / esum_b
        rvec = jnp.dot(onehot.T, a * out, preferred_element_type=jnp.float32)
        q_star = jnp.concatenate([q, rvec], axis=1)
    hidden = jax.nn.relu(
        jnp.dot(q_star, wo1_ref[...], preferred_element_type=jnp.float32)
        + bo1_ref[...])
    y_ref[...] = (jnp.dot(hidden, wo2_ref[...],
                          preferred_element_type=jnp.float32) + bo2_ref[...])


def _set2set(out, batch2d, wih, whh, bih, bhh, wo1, bo1, wo2, bo2):
    return pl.pallas_call(
        _set2set_body,
        out_shape=jax.ShapeDtypeStruct((B, 1), jnp.float32),
    )(out, batch2d, wih, whh, bih, bhh, wo1, bo1, wo2, bo2)


@functools.lru_cache(maxsize=None)
def _make_sc_gather():
    mesh = plsc.VectorSubcoreMesh(core_axis_name="c", subcore_axis_name="s")

    @functools.partial(
        pl.kernel, mesh=mesh,
        out_type=jax.ShapeDtypeStruct((E_PAD, OW), jnp.float32),
        scratch_types=[
            pltpu.VMEM((CHUNK,), jnp.int32),
            pltpu.VMEM((CHUNK, OW), jnp.float32),
            pltpu.SemaphoreType.DMA,
        ],
    )
    def gather_k(table_hbm, idx_hbm, out_hbm, idx_v, rows_v, sem):
        wid = lax.axis_index("s") * 2 + lax.axis_index("c")
        base = wid * EDGES_PER_TILE

        def body(j, carry):
            off = base + j * CHUNK
            pltpu.sync_copy(idx_hbm.at[pl.ds(off, CHUNK)], idx_v)
            pltpu.async_copy(table_hbm.at[idx_v], rows_v, sem).wait()
            pltpu.sync_copy(rows_v, out_hbm.at[pl.ds(off, CHUNK)])
            return carry

        lax.fori_loop(0, N_CHUNKS, body, 0)

    return gather_k


def _sc_gather(table, idx):
    return _make_sc_gather()(table, idx)


@functools.lru_cache(maxsize=None)
def _make_sc_scatter():
    mesh = plsc.VectorSubcoreMesh(core_axis_name="c", subcore_axis_name="s")

    @functools.partial(
        pl.kernel, mesh=mesh,
        out_type=jax.ShapeDtypeStruct((2, N_NODES, AW), jnp.float32),
        scratch_types=[
            pltpu.VMEM((CHUNK,), jnp.int32),
            pltpu.VMEM((CHUNK, AW), jnp.float32),
            pltpu.VMEM_SHARED((NPAD, AW), jnp.float32),
        ],
    )
    def scatter_k(msg_hbm, idx_hbm, zeros_hbm, out_hbm, idx_v, rows_v, acc_sh):
        cid = lax.axis_index("c")
        sid = lax.axis_index("s")
        wid = sid * 2 + cid

        rows_per_tile = NPAD // 16
        pltpu.sync_copy(zeros_hbm.at[pl.ds(sid * rows_per_tile, rows_per_tile)],
                        acc_sh.at[pl.ds(sid * rows_per_tile, rows_per_tile)])

        plsc.subcore_barrier()
        base = wid * EDGES_PER_TILE

        def body(j, carry):
            off = base + j * CHUNK
            pltpu.sync_copy(idx_hbm.at[pl.ds(off, CHUNK)], idx_v)
            pltpu.sync_copy(msg_hbm.at[pl.ds(off, CHUNK)], rows_v)
            pltpu.sync_copy(rows_v, acc_sh.at[idx_v], add=True)
            return carry

        lax.fori_loop(0, N_CHUNKS, body, 0)
        plsc.subcore_barrier()

        @pl.when(sid == 0)
        def _():
            pltpu.sync_copy(acc_sh.at[pl.ds(0, N_NODES)], out_hbm.at[cid])

    return scatter_k


def _sc_scatter(msg, idx, zeros):
    return _make_sc_scatter()(msg, idx, zeros)


def kernel(x, edge_index, edge_attr, batch, W_in, b_in, W_e1, b_e1, W_e2,
           b_e2, conv_bias, gru_W_ih, gru_W_hh, gru_b_ih, gru_b_hh,
           lstm_W_ih, lstm_W_hh, lstm_b_ih, lstm_b_hh, W_o1, b_o1, W_o2,
           b_o2):
    w_in_t = W_in.T
    b_in2 = b_in.reshape(1, S)
    w_e1_t = W_e1.T
    b_e12 = b_e1.reshape(1, S)
    gflat = W_e2.reshape(S, S, S).transpose(0, 2, 1).reshape(S * S, S)
    bmat = b_e2.reshape(S, S)
    eye = jnp.eye(S, dtype=jnp.float32)
    rep = jnp.kron(eye, jnp.ones((1, S), jnp.float32))
    tile = jnp.kron(jnp.ones((1, S), jnp.float32), eye)
    cb2 = conv_bias.reshape(1, S)
    wih_t = gru_W_ih.T
    whh_t = gru_W_hh.T
    bih2 = gru_b_ih.reshape(1, 3 * S)
    bhh2 = gru_b_hh.reshape(1, 3 * S)
    lwih_t = lstm_W_ih.T
    lwhh_t = lstm_W_hh.T
    lbih2 = lstm_b_ih.reshape(1, 4 * S)
    lbhh2 = lstm_b_hh.reshape(1, 4 * S)
    wo1_t = W_o1.T
    bo12 = b_o1.reshape(1, S)
    wo2_t = W_o2.T
    bo22 = b_o2.reshape(1, 1)

    pad_e = E_PAD - N_EDGES
    src = jnp.pad(edge_index[0], (0, pad_e))
    dst = jnp.pad(edge_index[1], (0, pad_e))
    ea_pad = jnp.pad(edge_attr, ((0, pad_e), (0, 0)))
    batch2d = batch.reshape(N_NODES, 1)
    zeros_acc = jnp.zeros((NPAD, AW), jnp.float32)

    out = _in_linear(x, w_in_t, b_in2)
    eh = _eh(ea_pad, w_e1_t, b_e12)

    h = out
    for _ in range(NUM_CONV):
        out_src = _sc_gather(out, src)
        msg = _msg(out_src, eh, gflat, bmat, rep, tile)
        partials = _sc_scatter(msg, dst, zeros_acc)
        h = _gru(partials, h, wih_t, whh_t, bih2, bhh2, cb2)
        out = h

    y = _set2set(out, batch2d, lwih_t, lwhh_t, lbih2, lbhh2,
                 wo1_t, bo12, wo2_t, bo22)
    return y.reshape(-1)


# speedup vs baseline: 1.9315x; 1.0697x over previous
import functools

import jax
import jax.numpy as jnp
from jax import lax
from jax.experimental import pallas as pl
from jax.experimental.pallas import tpu as pltpu
from jax.experimental.pallas import tpu_sc as plsc

N_NODES = 10000
N_EDGES = 160000
NODE_DIM = 128
EDGE_DIM = 16
S = 48
NUM_CONV = 3
B = 64

NW = 32
CHUNK = 128
E_PAD = 163840
EDGES_PER_TILE = E_PAD // NW
N_CHUNKS = EDGES_PER_TILE // CHUNK
AW = 128
OW = 128
NPAD = 10240


def _in_linear_body(x_ref, w_ref, b_ref, o_ref):
    o = jax.nn.relu(
        jnp.dot(x_ref[...], w_ref[...], preferred_element_type=jnp.float32)
        + b_ref[...])
    o_ref[...] = jnp.concatenate(
        [o, jnp.zeros((N_NODES, OW - S), jnp.float32)], axis=1)


def _in_linear(x, w, b):
    return pl.pallas_call(
        _in_linear_body,
        out_shape=jax.ShapeDtypeStruct((N_NODES, OW), jnp.float32),
    )(x, w, b)


def _eh_body(a_ref, w_ref, b_ref, o_ref):
    o_ref[...] = jax.nn.relu(
        jnp.dot(a_ref[...], w_ref[...], preferred_element_type=jnp.float32)
        + b_ref[...])


def _eh(edge_attr_pad, w, b):
    blk = 8192
    grid = E_PAD // blk
    return pl.pallas_call(
        _eh_body,
        grid=(grid,),
        in_specs=[
            pl.BlockSpec((blk, EDGE_DIM), lambda i: (i, 0)),
            pl.BlockSpec((EDGE_DIM, S), lambda i: (0, 0)),
            pl.BlockSpec((1, S), lambda i: (0, 0)),
        ],
        out_specs=pl.BlockSpec((blk, S), lambda i: (i, 0)),
        out_shape=jax.ShapeDtypeStruct((E_PAD, S), jnp.float32),
    )(edge_attr_pad, w, b)


_MSG_EB = 1280


def _msg_body(osrc_ref, eh_ref, g_ref, bm_ref, rep_ref, tile_ref, o_ref):
    osrc = osrc_ref[...][:, :S]
    ehb = eh_ref[...]
    osrc_rep = jnp.dot(osrc, rep_ref[...], preferred_element_type=jnp.float32)
    eh_tile = jnp.dot(ehb, tile_ref[...], preferred_element_type=jnp.float32)
    r = osrc_rep * eh_tile
    msg = jnp.dot(r, g_ref[...], preferred_element_type=jnp.float32)
    msg = msg + jnp.dot(osrc, bm_ref[...], preferred_element_type=jnp.float32)
    eidx = (pl.program_id(0) * _MSG_EB
            + lax.broadcasted_iota(jnp.int32, (_MSG_EB, 1), 0))
    valid = (eidx < N_EDGES).astype(jnp.float32)
    o_ref[...] = jnp.concatenate(
        [msg * valid, valid, jnp.zeros((_MSG_EB, AW - S - 1), jnp.float32)],
        axis=1)


def _msg(out_src, eh, gflat, bmat, rep, tile):
    grid = E_PAD // _MSG_EB
    return pl.pallas_call(
        _msg_body,
        grid=(grid,),
        in_specs=[
            pl.BlockSpec((_MSG_EB, OW), lambda i: (i, 0)),
            pl.BlockSpec((_MSG_EB, S), lambda i: (i, 0)),
            pl.BlockSpec((S * S, S), lambda i: (0, 0)),
            pl.BlockSpec((S, S), lambda i: (0, 0)),
            pl.BlockSpec((S, S * S), lambda i: (0, 0)),
            pl.BlockSpec((S, S * S), lambda i: (0, 0)),
        ],
        out_specs=pl.BlockSpec((_MSG_EB, AW), lambda i: (i, 0)),
        out_shape=jax.ShapeDtypeStruct((E_PAD, AW), jnp.float32),
    )(out_src, eh, gflat, bmat, rep, tile)


def _gru_body(p_ref, h_ref, wih_ref, whh_ref, bih_ref, bhh_ref, cb_ref, o_ref):
    p = p_ref[0] + p_ref[1]
    agg = p[:, :S]
    deg = jnp.maximum(p[:, S:S + 1], 1.0)
    m = jax.nn.relu(agg / deg + cb_ref[...])
    h = h_ref[...][:, :S]
    gi = jnp.dot(m, wih_ref[...], preferred_element_type=jnp.float32) + bih_ref[...]
    gh = jnp.dot(h, whh_ref[...], preferred_element_type=jnp.float32) + bhh_ref[...]
    r = jax.nn.sigmoid(gi[:, :S] + gh[:, :S])
    z = jax.nn.sigmoid(gi[:, S:2 * S] + gh[:, S:2 * S])
    n = jnp.tanh(gi[:, 2 * S:] + r * gh[:, 2 * S:])
    hn = (1.0 - z) * n + z * h
    o_ref[...] = jnp.concatenate(
        [hn, jnp.zeros((N_NODES, OW - S), jnp.float32)], axis=1)


def _gru(partials, h, wih, whh, bih, bhh, cb):
    return pl.pallas_call(
        _gru_body,
        out_shape=jax.ShapeDtypeStruct((N_NODES, OW), jnp.float32),
    )(partials, h, wih, whh, bih, bhh, cb)


def _set2set_body(out_ref, batch_ref, wih_ref, whh_ref, bih_ref, bhh_ref,
                  wo1_ref, bo1_ref, wo2_ref, bo2_ref, y_ref):
    out = out_ref[...][:, :S]
    bm = batch_ref[...]
    onehot = (bm == lax.broadcasted_iota(jnp.int32, (1, B), 1)
              ).astype(jnp.float32)
    q_star = jnp.zeros((B, 2 * S), jnp.float32)
    hs = jnp.zeros((B, S), jnp.float32)
    cs = jnp.zeros((B, S), jnp.float32)
    for _ in range(3):
        g = (jnp.dot(q_star, wih_ref[...], preferred_element_type=jnp.float32)
             + bih_ref[...]
             + jnp.dot(hs, whh_ref[...], preferred_element_type=jnp.float32)
             + bhh_ref[...])
        i = jax.nn.sigmoid(g[:, :S])
        f = jax.nn.sigmoid(g[:, S:2 * S])
        gc = jnp.tanh(g[:, 2 * S:3 * S])
        o = jax.nn.sigmoid(g[:, 3 * S:])
        cs = f * cs + i * gc
        hs = o * jnp.tanh(cs)
        q = hs
        qb = jnp.dot(onehot, q, preferred_element_type=jnp.float32)
        e = jnp.sum(out * qb, axis=-1, keepdims=True)
        emasked = jnp.where(onehot > 0.0, e, -jnp.inf)
        emax = jnp.max(emasked, axis=0, keepdims=True)
        emax_b = jnp.dot(onehot, emax.T, preferred_element_type=jnp.float32)
        ez = jnp.exp(e - emax_b)
        esum = jnp.dot(ez.T, onehot, preferred_element_type=jnp.float32)
        esum_b = jnp.dot(onehot, esum.T, preferred_element_type=jnp.float32)
        a = ez / esum_b
        rvec = jnp.dot(onehot.T, a * out, preferred_element_type=jnp.float32)
        q_star = jnp.concatenate([q, rvec], axis=1)
    hidden = jax.nn.relu(
        jnp.dot(q_star, wo1_ref[...], preferred_element_type=jnp.float32)
        + bo1_ref[...])
    y_ref[...] = (jnp.dot(hidden, wo2_ref[...],
                          preferred_element_type=jnp.float32) + bo2_ref[...])


def _set2set(out, batch2d, wih, whh, bih, bhh, wo1, bo1, wo2, bo2):
    return pl.pallas_call(
        _set2set_body,
        out_shape=jax.ShapeDtypeStruct((B, 1), jnp.float32),
    )(out, batch2d, wih, whh, bih, bhh, wo1, bo1, wo2, bo2)


@functools.lru_cache(maxsize=None)
def _make_sc_gather():
    mesh = plsc.VectorSubcoreMesh(core_axis_name="c", subcore_axis_name="s")

    @functools.partial(
        pl.kernel, mesh=mesh,
        out_type=jax.ShapeDtypeStruct((E_PAD, OW), jnp.float32),
        scratch_types=[
            pltpu.VMEM((N_CHUNKS, CHUNK), jnp.int32),
            pltpu.VMEM((CHUNK, OW), jnp.float32),
            pltpu.VMEM((CHUNK, OW), jnp.float32),
            pltpu.SemaphoreType.DMA,
            pltpu.SemaphoreType.DMA,
            pltpu.SemaphoreType.DMA,
            pltpu.SemaphoreType.DMA,
        ],
    )
    def gather_k(table_hbm, idx_hbm3, out_hbm, idx_all, rows0, rows1,
                 gs0, gs1, ws0, ws1):
        wid = lax.axis_index("s") * 2 + lax.axis_index("c")
        base = wid * EDGES_PER_TILE
        pltpu.sync_copy(idx_hbm3.at[wid], idx_all)
        rows = (rows0, rows1)
        gsem = (gs0, gs1)
        wsem = (ws0, ws1)

        def pair(p, carry):
            handles = []
            for b in range(2):
                j = p * 2 + b

                @pl.when(p >= 1)
                def _(b=b):
                    pltpu.make_async_copy(
                        rows[b], out_hbm.at[pl.ds(base, CHUNK)], wsem[b]
                    ).wait()

                handles.append(pltpu.async_copy(
                    table_hbm.at[idx_all.at[j]], rows[b], gsem[b]))
            for b in range(2):
                j = p * 2 + b
                handles[b].wait()
                pltpu.async_copy(
                    rows[b], out_hbm.at[pl.ds(base + j * CHUNK, CHUNK)],
                    wsem[b])
            return carry

        lax.fori_loop(0, N_CHUNKS // 2, pair, 0)
        for b in range(2):
            pltpu.make_async_copy(
                rows[b], out_hbm.at[pl.ds(base, CHUNK)], wsem[b]).wait()

    return gather_k


def _sc_gather(table, idx3):
    return _make_sc_gather()(table, idx3)


@functools.lru_cache(maxsize=None)
def _make_sc_scatter():
    mesh = plsc.VectorSubcoreMesh(core_axis_name="c", subcore_axis_name="s")

    @functools.partial(
        pl.kernel, mesh=mesh,
        out_type=jax.ShapeDtypeStruct((2, N_NODES, AW), jnp.float32),
        scratch_types=[
            pltpu.VMEM((N_CHUNKS, CHUNK), jnp.int32),
            pltpu.VMEM((CHUNK, AW), jnp.float32),
            pltpu.VMEM((CHUNK, AW), jnp.float32),
            pltpu.VMEM_SHARED((NPAD, AW), jnp.float32),
            pltpu.SemaphoreType.DMA,
            pltpu.SemaphoreType.DMA,
            pltpu.SemaphoreType.DMA,
            pltpu.SemaphoreType.DMA,
        ],
    )
    def scatter_k(msg_hbm, idx_hbm3, zeros_hbm, out_hbm, idx_all,
                  rows0, rows1, acc_sh, ls0, ls1, ss0, ss1):
        cid = lax.axis_index("c")
        sid = lax.axis_index("s")
        wid = sid * 2 + cid

        rows_per_tile = NPAD // 16
        pltpu.sync_copy(zeros_hbm.at[pl.ds(sid * rows_per_tile, rows_per_tile)],
                        acc_sh.at[pl.ds(sid * rows_per_tile, rows_per_tile)])
        base = wid * EDGES_PER_TILE
        pltpu.sync_copy(idx_hbm3.at[wid], idx_all)

        plsc.subcore_barrier()
        rows = (rows0, rows1)
        lsem = (ls0, ls1)
        ssem = (ss0, ss1)

        def pair(p, carry):
            handles = []
            for b in range(2):
                j = p * 2 + b

                @pl.when(p >= 1)
                def _(b=b):
                    pltpu.make_async_copy(
                        rows[b], acc_sh.at[idx_all.at[0]], ssem[b]).wait()

                handles.append(pltpu.async_copy(
                    msg_hbm.at[pl.ds(base + j * CHUNK, CHUNK)], rows[b],
                    lsem[b]))
            for b in range(2):
                j = p * 2 + b
                handles[b].wait()
                pltpu.async_copy(rows[b], acc_sh.at[idx_all.at[j]],
                                 ssem[b], add=True)
            return carry

        lax.fori_loop(0, N_CHUNKS // 2, pair, 0)
        for b in range(2):
            pltpu.make_async_copy(
                rows[b], acc_sh.at[idx_all.at[0]], ssem[b]).wait()
        plsc.subcore_barrier()

        @pl.when(sid == 0)
        def _():
            pltpu.sync_copy(acc_sh.at[pl.ds(0, N_NODES)], out_hbm.at[cid])

    return scatter_k


def _sc_scatter(msg, idx, zeros):
    return _make_sc_scatter()(msg, idx, zeros)


def kernel(x, edge_index, edge_attr, batch, W_in, b_in, W_e1, b_e1, W_e2,
           b_e2, conv_bias, gru_W_ih, gru_W_hh, gru_b_ih, gru_b_hh,
           lstm_W_ih, lstm_W_hh, lstm_b_ih, lstm_b_hh, W_o1, b_o1, W_o2,
           b_o2):
    w_in_t = W_in.T
    b_in2 = b_in.reshape(1, S)
    w_e1_t = W_e1.T
    b_e12 = b_e1.reshape(1, S)
    gflat = W_e2.reshape(S, S, S).transpose(0, 2, 1).reshape(S * S, S)
    bmat = b_e2.reshape(S, S)
    eye = jnp.eye(S, dtype=jnp.float32)
    rep = jnp.kron(eye, jnp.ones((1, S), jnp.float32))
    tile = jnp.kron(jnp.ones((1, S), jnp.float32), eye)
    cb2 = conv_bias.reshape(1, S)
    wih_t = gru_W_ih.T
    whh_t = gru_W_hh.T
    bih2 = gru_b_ih.reshape(1, 3 * S)
    bhh2 = gru_b_hh.reshape(1, 3 * S)
    lwih_t = lstm_W_ih.T
    lwhh_t = lstm_W_hh.T
    lbih2 = lstm_b_ih.reshape(1, 4 * S)
    lbhh2 = lstm_b_hh.reshape(1, 4 * S)
    wo1_t = W_o1.T
    bo12 = b_o1.reshape(1, S)
    wo2_t = W_o2.T
    bo22 = b_o2.reshape(1, 1)

    pad_e = E_PAD - N_EDGES
    src = jnp.pad(edge_index[0], (0, pad_e)).reshape(NW, N_CHUNKS, CHUNK)
    dst = jnp.pad(edge_index[1], (0, pad_e)).reshape(NW, N_CHUNKS, CHUNK)
    ea_pad = jnp.pad(edge_attr, ((0, pad_e), (0, 0)))
    batch2d = batch.reshape(N_NODES, 1)
    zeros_acc = jnp.zeros((NPAD, AW), jnp.float32)

    out = _in_linear(x, w_in_t, b_in2)
    eh = _eh(ea_pad, w_e1_t, b_e12)

    h = out
    for _ in range(NUM_CONV):
        out_src = _sc_gather(out, src)
        msg = _msg(out_src, eh, gflat, bmat, rep, tile)
        partials = _sc_scatter(msg, dst, zeros_acc)
        h = _gru(partials, h, wih_t, whh_t, bih2, bhh2, cb2)
        out = h

    y = _set2set(out, batch2d, lwih_t, lwhh_t, lbih2, lbhh2,
                 wo1_t, bo12, wo2_t, bo22)
    return y.reshape(-1)


# speedup vs baseline: 2.2869x; 1.1840x over previous
import functools

import jax
import jax.numpy as jnp
from jax import lax
from jax.experimental import pallas as pl
from jax.experimental.pallas import tpu as pltpu
from jax.experimental.pallas import tpu_sc as plsc

N_NODES = 10000
N_EDGES = 160000
NODE_DIM = 128
EDGE_DIM = 16
S = 48
NUM_CONV = 3
B = 64

NW = 32
CHUNK = 128
E_PAD = 163840
EDGES_PER_TILE = E_PAD // NW
N_CHUNKS = EDGES_PER_TILE // CHUNK
AW = 128
OW = 128
NPAD = 10240


def _in_linear_body(x_ref, w_ref, b_ref, o_ref):
    o = jax.nn.relu(
        jnp.dot(x_ref[...], w_ref[...], preferred_element_type=jnp.float32)
        + b_ref[...])
    o_ref[...] = jnp.concatenate(
        [o, jnp.zeros((N_NODES, OW - S), jnp.float32)], axis=1)


def _in_linear(x, w, b):
    return pl.pallas_call(
        _in_linear_body,
        out_shape=jax.ShapeDtypeStruct((N_NODES, OW), jnp.float32),
    )(x, w, b)


def _eh_body(a_ref, w_ref, b_ref, o_ref):
    o_ref[...] = jax.nn.relu(
        jnp.dot(a_ref[...], w_ref[...], preferred_element_type=jnp.float32)
        + b_ref[...])


def _eh(edge_attr_pad, w, b):
    blk = 8192
    grid = E_PAD // blk
    return pl.pallas_call(
        _eh_body,
        grid=(grid,),
        in_specs=[
            pl.BlockSpec((blk, EDGE_DIM), lambda i: (i, 0)),
            pl.BlockSpec((EDGE_DIM, S), lambda i: (0, 0)),
            pl.BlockSpec((1, S), lambda i: (0, 0)),
        ],
        out_specs=pl.BlockSpec((blk, S), lambda i: (i, 0)),
        out_shape=jax.ShapeDtypeStruct((E_PAD, S), jnp.float32),
    )(edge_attr_pad, w, b)


_MSG_EB = 1280


def _msg_body(off0, osrc_ref, eh_ref, g_ref, bm_ref, rep_ref, tile_ref, o_ref):
    osrc = osrc_ref[...][:, :S]
    ehb = eh_ref[...]
    osrc_rep = jnp.dot(osrc, rep_ref[...], preferred_element_type=jnp.float32)
    eh_tile = jnp.dot(ehb, tile_ref[...], preferred_element_type=jnp.float32)
    r = osrc_rep * eh_tile
    msg = jnp.dot(r, g_ref[...], preferred_element_type=jnp.float32)
    msg = msg + jnp.dot(osrc, bm_ref[...], preferred_element_type=jnp.float32)
    eidx = (off0 + pl.program_id(0) * _MSG_EB
            + lax.broadcasted_iota(jnp.int32, (_MSG_EB, 1), 0))
    valid = (eidx < N_EDGES).astype(jnp.float32)
    o_ref[...] = jnp.concatenate(
        [msg * valid, valid, jnp.zeros((_MSG_EB, AW - S - 1), jnp.float32)],
        axis=1)


def _msg(out_src, eh, gflat, bmat, rep, tile, off0):
    grid = out_src.shape[0] // _MSG_EB
    return pl.pallas_call(
        functools.partial(_msg_body, off0),
        grid=(grid,),
        in_specs=[
            pl.BlockSpec((_MSG_EB, OW), lambda i: (i, 0)),
            pl.BlockSpec((_MSG_EB, S), lambda i: (i, 0)),
            pl.BlockSpec((S * S, S), lambda i: (0, 0)),
            pl.BlockSpec((S, S), lambda i: (0, 0)),
            pl.BlockSpec((S, S * S), lambda i: (0, 0)),
            pl.BlockSpec((S, S * S), lambda i: (0, 0)),
        ],
        out_specs=pl.BlockSpec((_MSG_EB, AW), lambda i: (i, 0)),
        out_shape=jax.ShapeDtypeStruct((out_src.shape[0], AW), jnp.float32),
    )(out_src, eh, gflat, bmat, rep, tile)


def _gru_body(p_ref, p2_ref, h_ref, wih_ref, whh_ref, bih_ref, bhh_ref,
              cb_ref, o_ref):
    p = (p_ref[0] + p_ref[1]) + (p2_ref[0] + p2_ref[1])
    agg = p[:, :S]
    deg = jnp.maximum(p[:, S:S + 1], 1.0)
    m = jax.nn.relu(agg / deg + cb_ref[...])
    h = h_ref[...][:, :S]
    gi = jnp.dot(m, wih_ref[...], preferred_element_type=jnp.float32) + bih_ref[...]
    gh = jnp.dot(h, whh_ref[...], preferred_element_type=jnp.float32) + bhh_ref[...]
    r = jax.nn.sigmoid(gi[:, :S] + gh[:, :S])
    z = jax.nn.sigmoid(gi[:, S:2 * S] + gh[:, S:2 * S])
    n = jnp.tanh(gi[:, 2 * S:] + r * gh[:, 2 * S:])
    hn = (1.0 - z) * n + z * h
    o_ref[...] = jnp.concatenate(
        [hn, jnp.zeros((N_NODES, OW - S), jnp.float32)], axis=1)


def _gru(partials, partials2, h, wih, whh, bih, bhh, cb):
    return pl.pallas_call(
        _gru_body,
        out_shape=jax.ShapeDtypeStruct((N_NODES, OW), jnp.float32),
    )(partials, partials2, h, wih, whh, bih, bhh, cb)


def _set2set_body(out_ref, batch_ref, wih_ref, whh_ref, bih_ref, bhh_ref,
                  wo1_ref, bo1_ref, wo2_ref, bo2_ref, y_ref):
    out = out_ref[...][:, :S]
    bm = batch_ref[...]
    onehot = (bm == lax.broadcasted_iota(jnp.int32, (1, B), 1)
              ).astype(jnp.float32)
    q_star = jnp.zeros((B, 2 * S), jnp.float32)
    hs = jnp.zeros((B, S), jnp.float32)
    cs = jnp.zeros((B, S), jnp.float32)
    for _ in range(3):
        g = (jnp.dot(q_star, wih_ref[...], preferred_element_type=jnp.float32)
             + bih_ref[...]
             + jnp.dot(hs, whh_ref[...], preferred_element_type=jnp.float32)
             + bhh_ref[...])
        i = jax.nn.sigmoid(g[:, :S])
        f = jax.nn.sigmoid(g[:, S:2 * S])
        gc = jnp.tanh(g[:, 2 * S:3 * S])
        o = jax.nn.sigmoid(g[:, 3 * S:])
        cs = f * cs + i * gc
        hs = o * jnp.tanh(cs)
        q = hs
        qb = jnp.dot(onehot, q, preferred_element_type=jnp.float32)
        e = jnp.sum(out * qb, axis=-1, keepdims=True)
        emasked = jnp.where(onehot > 0.0, e, -jnp.inf)
        emax = jnp.max(emasked, axis=0, keepdims=True)
        emax_b = jnp.dot(onehot, emax.T, preferred_element_type=jnp.float32)
        ez = jnp.exp(e - emax_b)
        esum = jnp.dot(ez.T, onehot, preferred_element_type=jnp.float32)
        esum_b = jnp.dot(onehot, esum.T, preferred_element_type=jnp.float32)
        a = ez / esum_b
        rvec = jnp.dot(onehot.T, a * out, preferred_element_type=jnp.float32)
        q_star = jnp.concatenate([q, rvec], axis=1)
    hidden = jax.nn.relu(
        jnp.dot(q_star, wo1_ref[...], preferred_element_type=jnp.float32)
        + bo1_ref[...])
    y_ref[...] = (jnp.dot(hidden, wo2_ref[...],
                          preferred_element_type=jnp.float32) + bo2_ref[...])


def _set2set(out, batch2d, wih, whh, bih, bhh, wo1, bo1, wo2, bo2):
    return pl.pallas_call(
        _set2set_body,
        out_shape=jax.ShapeDtypeStruct((B, 1), jnp.float32),
    )(out, batch2d, wih, whh, bih, bhh, wo1, bo1, wo2, bo2)


@functools.lru_cache(maxsize=None)
def _make_sc_gather(n_chunks):
    n_edges_tile = n_chunks * CHUNK
    mesh = plsc.VectorSubcoreMesh(core_axis_name="c", subcore_axis_name="s")

    @functools.partial(
        pl.kernel, mesh=mesh,
        out_type=jax.ShapeDtypeStruct((NW * n_edges_tile, OW), jnp.float32),
        scratch_types=[
            pltpu.VMEM((n_chunks, CHUNK), jnp.int32),
            pltpu.VMEM((CHUNK, OW), jnp.float32),
            pltpu.VMEM((CHUNK, OW), jnp.float32),
            pltpu.SemaphoreType.DMA,
            pltpu.SemaphoreType.DMA,
            pltpu.SemaphoreType.DMA,
            pltpu.SemaphoreType.DMA,
        ],
    )
    def gather_k(table_hbm, idx_hbm3, out_hbm, idx_all, rows0, rows1,
                 gs0, gs1, ws0, ws1):
        wid = lax.axis_index("s") * 2 + lax.axis_index("c")
        base = wid * n_edges_tile
        pltpu.sync_copy(idx_hbm3.at[wid], idx_all)
        rows = (rows0, rows1)
        gsem = (gs0, gs1)
        wsem = (ws0, ws1)

        def pair(p, carry):
            handles = []
            for b in range(2):
                j = p * 2 + b

                @pl.when(p >= 1)
                def _(b=b):
                    pltpu.make_async_copy(
                        rows[b], out_hbm.at[pl.ds(base, CHUNK)], wsem[b]
                    ).wait()

                handles.append(pltpu.async_copy(
                    table_hbm.at[idx_all.at[j]], rows[b], gsem[b]))
            for b in range(2):
                j = p * 2 + b
                handles[b].wait()
                pltpu.async_copy(
                    rows[b], out_hbm.at[pl.ds(base + j * CHUNK, CHUNK)],
                    wsem[b])
            return carry

        lax.fori_loop(0, n_chunks // 2, pair, 0)
        for b in range(2):
            pltpu.make_async_copy(
                rows[b], out_hbm.at[pl.ds(base, CHUNK)], wsem[b]).wait()

    return gather_k


def _sc_gather(table, idx3):
    return _make_sc_gather(idx3.shape[1])(table, idx3)


@functools.lru_cache(maxsize=None)
def _make_sc_scatter(n_chunks):
    n_edges_tile = n_chunks * CHUNK
    mesh = plsc.VectorSubcoreMesh(core_axis_name="c", subcore_axis_name="s")

    @functools.partial(
        pl.kernel, mesh=mesh,
        out_type=jax.ShapeDtypeStruct((2, N_NODES, AW), jnp.float32),
        scratch_types=[
            pltpu.VMEM((n_chunks, CHUNK), jnp.int32),
            pltpu.VMEM((CHUNK, AW), jnp.float32),
            pltpu.VMEM((CHUNK, AW), jnp.float32),
            pltpu.VMEM_SHARED((NPAD, AW), jnp.float32),
            pltpu.SemaphoreType.DMA,
            pltpu.SemaphoreType.DMA,
            pltpu.SemaphoreType.DMA,
            pltpu.SemaphoreType.DMA,
        ],
    )
    def scatter_k(msg_hbm, idx_hbm3, zeros_hbm, out_hbm, idx_all,
                  rows0, rows1, acc_sh, ls0, ls1, ss0, ss1):
        cid = lax.axis_index("c")
        sid = lax.axis_index("s")
        wid = sid * 2 + cid

        rows_per_tile = NPAD // 16
        pltpu.sync_copy(zeros_hbm.at[pl.ds(sid * rows_per_tile, rows_per_tile)],
                        acc_sh.at[pl.ds(sid * rows_per_tile, rows_per_tile)])
        base = wid * n_edges_tile
        pltpu.sync_copy(idx_hbm3.at[wid], idx_all)

        plsc.subcore_barrier()
        rows = (rows0, rows1)
        lsem = (ls0, ls1)
        ssem = (ss0, ss1)

        def pair(p, carry):
            handles = []
            for b in range(2):
                j = p * 2 + b

                @pl.when(p >= 1)
                def _(b=b):
                    pltpu.make_async_copy(
                        rows[b], acc_sh.at[idx_all.at[0]], ssem[b]).wait()

                handles.append(pltpu.async_copy(
                    msg_hbm.at[pl.ds(base + j * CHUNK, CHUNK)], rows[b],
                    lsem[b]))
            for b in range(2):
                j = p * 2 + b
                handles[b].wait()
                pltpu.async_copy(rows[b], acc_sh.at[idx_all.at[j]],
                                 ssem[b], add=True)
            return carry

        lax.fori_loop(0, n_chunks // 2, pair, 0)
        for b in range(2):
            pltpu.make_async_copy(
                rows[b], acc_sh.at[idx_all.at[0]], ssem[b]).wait()
        plsc.subcore_barrier()

        @pl.when(sid == 0)
        def _():
            pltpu.sync_copy(acc_sh.at[pl.ds(0, N_NODES)], out_hbm.at[cid])

    return scatter_k


def _sc_scatter(msg, idx3, zeros):
    return _make_sc_scatter(idx3.shape[1])(msg, idx3, zeros)


def kernel(x, edge_index, edge_attr, batch, W_in, b_in, W_e1, b_e1, W_e2,
           b_e2, conv_bias, gru_W_ih, gru_W_hh, gru_b_ih, gru_b_hh,
           lstm_W_ih, lstm_W_hh, lstm_b_ih, lstm_b_hh, W_o1, b_o1, W_o2,
           b_o2):
    w_in_t = W_in.T
    b_in2 = b_in.reshape(1, S)
    w_e1_t = W_e1.T
    b_e12 = b_e1.reshape(1, S)
    gflat = W_e2.reshape(S, S, S).transpose(0, 2, 1).reshape(S * S, S)
    bmat = b_e2.reshape(S, S)
    eye = jnp.eye(S, dtype=jnp.float32)
    rep = jnp.kron(eye, jnp.ones((1, S), jnp.float32))
    tile = jnp.kron(jnp.ones((1, S), jnp.float32), eye)
    cb2 = conv_bias.reshape(1, S)
    wih_t = gru_W_ih.T
    whh_t = gru_W_hh.T
    bih2 = gru_b_ih.reshape(1, 3 * S)
    bhh2 = gru_b_hh.reshape(1, 3 * S)
    lwih_t = lstm_W_ih.T
    lwhh_t = lstm_W_hh.T
    lbih2 = lstm_b_ih.reshape(1, 4 * S)
    lbhh2 = lstm_b_hh.reshape(1, 4 * S)
    wo1_t = W_o1.T
    bo12 = b_o1.reshape(1, S)
    wo2_t = W_o2.T
    bo22 = b_o2.reshape(1, 1)

    pad_e = E_PAD - N_EDGES
    e_half = E_PAD // 2
    nch = N_CHUNKS // 2
    src_p = jnp.pad(edge_index[0], (0, pad_e))
    dst_p = jnp.pad(edge_index[1], (0, pad_e))
    src_h = [src_p[:e_half].reshape(NW, nch, CHUNK),
             src_p[e_half:].reshape(NW, nch, CHUNK)]
    dst_h = [dst_p[:e_half].reshape(NW, nch, CHUNK),
             dst_p[e_half:].reshape(NW, nch, CHUNK)]
    ea_pad = jnp.pad(edge_attr, ((0, pad_e), (0, 0)))
    batch2d = batch.reshape(N_NODES, 1)
    zeros_acc = jnp.zeros((NPAD, AW), jnp.float32)

    out = _in_linear(x, w_in_t, b_in2)
    eh = _eh(ea_pad, w_e1_t, b_e12)

    eh_h = [eh[:e_half], eh[e_half:]]
    h = out
    for _ in range(NUM_CONV):
        os0 = _sc_gather(out, src_h[0])
        os1 = _sc_gather(out, src_h[1])
        m0 = _msg(os0, eh_h[0], gflat, bmat, rep, tile, 0)
        m1 = _msg(os1, eh_h[1], gflat, bmat, rep, tile, e_half)
        p0 = _sc_scatter(m0, dst_h[0], zeros_acc)
        p1 = _sc_scatter(m1, dst_h[1], zeros_acc)
        h = _gru(p0, p1, h, wih_t, whh_t, bih2, bhh2, cb2)
        out = h

    y = _set2set(out, batch2d, lwih_t, lwhh_t, lbih2, lbhh2,
                 wo1_t, bo12, wo2_t, bo22)
    return y.reshape(-1)


# speedup vs baseline: 3.1164x; 1.3627x over previous
import functools

import jax
import jax.numpy as jnp
from jax import lax
from jax.experimental import pallas as pl
from jax.experimental.pallas import tpu as pltpu
from jax.experimental.pallas import tpu_sc as plsc

N_NODES = 10000
N_EDGES = 160000
NODE_DIM = 128
EDGE_DIM = 16
S = 48
NUM_CONV = 3
B = 64

NW = 32
CHUNK = 128
E_PAD = 163840
EDGES_PER_TILE = E_PAD // NW
N_CHUNKS = EDGES_PER_TILE // CHUNK
AW = 128
OW = 128
NPAD = 10240


def _in_linear_body(x_ref, w_ref, b_ref, o_ref):
    o = jax.nn.relu(
        jnp.dot(x_ref[...], w_ref[...], preferred_element_type=jnp.float32)
        + b_ref[...])
    o_ref[...] = jnp.concatenate(
        [o, jnp.zeros((N_NODES, OW - S), jnp.float32)], axis=1)


def _in_linear(x, w, b):
    return pl.pallas_call(
        _in_linear_body,
        out_shape=jax.ShapeDtypeStruct((N_NODES, OW), jnp.float32),
    )(x, w, b)


def _eh_body(a_ref, w_ref, b_ref, o_ref):
    o_ref[...] = jax.nn.relu(
        jnp.dot(a_ref[...], w_ref[...], preferred_element_type=jnp.float32)
        + b_ref[...])


def _eh(edge_attr_pad, w, b):
    blk = 8192
    grid = E_PAD // blk
    return pl.pallas_call(
        _eh_body,
        grid=(grid,),
        in_specs=[
            pl.BlockSpec((blk, EDGE_DIM), lambda i: (i, 0)),
            pl.BlockSpec((EDGE_DIM, S), lambda i: (0, 0)),
            pl.BlockSpec((1, S), lambda i: (0, 0)),
        ],
        out_specs=pl.BlockSpec((blk, S), lambda i: (i, 0)),
        out_shape=jax.ShapeDtypeStruct((E_PAD, S), jnp.float32),
    )(edge_attr_pad, w, b)


_MSG_EB = 1280


def _msg_body(off0, osrc_ref, eh_ref, g_ref, bm_ref, rep_ref, tile_ref, o_ref):
    osrc = osrc_ref[...][:, :S]
    ehb = eh_ref[...]
    osrc_b = osrc.astype(jnp.bfloat16)
    ehb_b = ehb.astype(jnp.bfloat16)
    osrc_rep = jnp.dot(osrc_b, rep_ref[...], preferred_element_type=jnp.float32)
    eh_tile = jnp.concatenate([ehb] * S, axis=1)
    r = (osrc_rep * eh_tile).astype(jnp.bfloat16)
    msg = jnp.dot(r, g_ref[...], preferred_element_type=jnp.float32)
    msg = msg + jnp.dot(osrc, bm_ref[...], preferred_element_type=jnp.float32)
    eidx = (off0 + pl.program_id(0) * _MSG_EB
            + lax.broadcasted_iota(jnp.int32, (_MSG_EB, 1), 0))
    valid = (eidx < N_EDGES).astype(jnp.float32)
    o_ref[...] = jnp.concatenate(
        [msg * valid, valid, jnp.zeros((_MSG_EB, AW - S - 1), jnp.float32)],
        axis=1)


def _msg(out_src, eh, gflat, bmat, rep, tile, off0):
    grid = out_src.shape[0] // _MSG_EB
    return pl.pallas_call(
        functools.partial(_msg_body, off0),
        grid=(grid,),
        in_specs=[
            pl.BlockSpec((_MSG_EB, OW), lambda i: (i, 0)),
            pl.BlockSpec((_MSG_EB, S), lambda i: (i, 0)),
            pl.BlockSpec((S * S, S), lambda i: (0, 0)),
            pl.BlockSpec((S, S), lambda i: (0, 0)),
            pl.BlockSpec((S, S * S), lambda i: (0, 0)),
            pl.BlockSpec((S, S * S), lambda i: (0, 0)),
        ],
        out_specs=pl.BlockSpec((_MSG_EB, AW), lambda i: (i, 0)),
        out_shape=jax.ShapeDtypeStruct((out_src.shape[0], AW), jnp.float32),
    )(out_src, eh, gflat, bmat, rep, tile)


def _gru_body(p_ref, p2_ref, h_ref, wih_ref, whh_ref, bih_ref, bhh_ref,
              cb_ref, o_ref):
    p = (p_ref[0] + p_ref[1]) + (p2_ref[0] + p2_ref[1])
    agg = p[:, :S]
    deg = jnp.maximum(p[:, S:S + 1], 1.0)
    m = jax.nn.relu(agg / deg + cb_ref[...])
    h = h_ref[...][:, :S]
    gi = jnp.dot(m, wih_ref[...], preferred_element_type=jnp.float32) + bih_ref[...]
    gh = jnp.dot(h, whh_ref[...], preferred_element_type=jnp.float32) + bhh_ref[...]
    r = jax.nn.sigmoid(gi[:, :S] + gh[:, :S])
    z = jax.nn.sigmoid(gi[:, S:2 * S] + gh[:, S:2 * S])
    n = jnp.tanh(gi[:, 2 * S:] + r * gh[:, 2 * S:])
    hn = (1.0 - z) * n + z * h
    o_ref[...] = jnp.concatenate(
        [hn, jnp.zeros((N_NODES, OW - S), jnp.float32)], axis=1)


def _gru(partials, partials2, h, wih, whh, bih, bhh, cb):
    return pl.pallas_call(
        _gru_body,
        out_shape=jax.ShapeDtypeStruct((N_NODES, OW), jnp.float32),
    )(partials, partials2, h, wih, whh, bih, bhh, cb)


def _set2set_body(out_ref, batch_ref, wih_ref, whh_ref, bih_ref, bhh_ref,
                  wo1_ref, bo1_ref, wo2_ref, bo2_ref, y_ref):
    out = out_ref[...][:, :S]
    bm = batch_ref[...]
    onehot = (bm == lax.broadcasted_iota(jnp.int32, (1, B), 1)
              ).astype(jnp.float32)
    q_star = jnp.zeros((B, 2 * S), jnp.float32)
    hs = jnp.zeros((B, S), jnp.float32)
    cs = jnp.zeros((B, S), jnp.float32)
    for _ in range(3):
        g = (jnp.dot(q_star, wih_ref[...], preferred_element_type=jnp.float32)
             + bih_ref[...]
             + jnp.dot(hs, whh_ref[...], preferred_element_type=jnp.float32)
             + bhh_ref[...])
        i = jax.nn.sigmoid(g[:, :S])
        f = jax.nn.sigmoid(g[:, S:2 * S])
        gc = jnp.tanh(g[:, 2 * S:3 * S])
        o = jax.nn.sigmoid(g[:, 3 * S:])
        cs = f * cs + i * gc
        hs = o * jnp.tanh(cs)
        q = hs
        qb = jnp.dot(onehot, q, preferred_element_type=jnp.float32)
        e = jnp.sum(out * qb, axis=-1, keepdims=True)
        emasked = jnp.where(onehot > 0.0, e, -jnp.inf)
        emax = jnp.max(emasked, axis=0, keepdims=True)
        emax_b = jnp.dot(onehot, emax.T, preferred_element_type=jnp.float32)
        ez = jnp.exp(e - emax_b)
        esum = jnp.dot(ez.T, onehot, preferred_element_type=jnp.float32)
        esum_b = jnp.dot(onehot, esum.T, preferred_element_type=jnp.float32)
        a = ez / esum_b
        rvec = jnp.dot(onehot.T, a * out, preferred_element_type=jnp.float32)
        q_star = jnp.concatenate([q, rvec], axis=1)
    hidden = jax.nn.relu(
        jnp.dot(q_star, wo1_ref[...], preferred_element_type=jnp.float32)
        + bo1_ref[...])
    y_ref[...] = (jnp.dot(hidden, wo2_ref[...],
                          preferred_element_type=jnp.float32) + bo2_ref[...])


def _set2set(out, batch2d, wih, whh, bih, bhh, wo1, bo1, wo2, bo2):
    return pl.pallas_call(
        _set2set_body,
        out_shape=jax.ShapeDtypeStruct((B, 1), jnp.float32),
    )(out, batch2d, wih, whh, bih, bhh, wo1, bo1, wo2, bo2)


@functools.lru_cache(maxsize=None)
def _make_sc_gather(n_chunks):
    n_edges_tile = n_chunks * CHUNK
    mesh = plsc.VectorSubcoreMesh(core_axis_name="c", subcore_axis_name="s")

    @functools.partial(
        pl.kernel, mesh=mesh,
        out_type=jax.ShapeDtypeStruct((NW * n_edges_tile, OW), jnp.float32),
        scratch_types=[
            pltpu.VMEM((n_chunks, CHUNK), jnp.int32),
            pltpu.VMEM((CHUNK, OW), jnp.float32),
            pltpu.VMEM((CHUNK, OW), jnp.float32),
            pltpu.SemaphoreType.DMA,
            pltpu.SemaphoreType.DMA,
            pltpu.SemaphoreType.DMA,
            pltpu.SemaphoreType.DMA,
        ],
    )
    def gather_k(table_hbm, idx_hbm3, out_hbm, idx_all, rows0, rows1,
                 gs0, gs1, ws0, ws1):
        wid = lax.axis_index("s") * 2 + lax.axis_index("c")
        base = wid * n_edges_tile
        pltpu.sync_copy(idx_hbm3.at[wid], idx_all)
        rows = (rows0, rows1)
        gsem = (gs0, gs1)
        wsem = (ws0, ws1)

        def pair(p, carry):
            handles = []
            for b in range(2):
                j = p * 2 + b

                @pl.when(p >= 1)
                def _(b=b):
                    pltpu.make_async_copy(
                        rows[b], out_hbm.at[pl.ds(base, CHUNK)], wsem[b]
                    ).wait()

                handles.append(pltpu.async_copy(
                    table_hbm.at[idx_all.at[j]], rows[b], gsem[b]))
            for b in range(2):
                j = p * 2 + b
                handles[b].wait()
                pltpu.async_copy(
                    rows[b], out_hbm.at[pl.ds(base + j * CHUNK, CHUNK)],
                    wsem[b])
            return carry

        lax.fori_loop(0, n_chunks // 2, pair, 0)
        for b in range(2):
            pltpu.make_async_copy(
                rows[b], out_hbm.at[pl.ds(base, CHUNK)], wsem[b]).wait()

    return gather_k


def _sc_gather(table, idx3):
    return _make_sc_gather(idx3.shape[1])(table, idx3)


@functools.lru_cache(maxsize=None)
def _make_sc_scatter(n_chunks):
    n_edges_tile = n_chunks * CHUNK
    mesh = plsc.VectorSubcoreMesh(core_axis_name="c", subcore_axis_name="s")

    @functools.partial(
        pl.kernel, mesh=mesh,
        out_type=jax.ShapeDtypeStruct((2, N_NODES, AW), jnp.float32),
        scratch_types=[
            pltpu.VMEM((n_chunks, CHUNK), jnp.int32),
            pltpu.VMEM((CHUNK, AW), jnp.float32),
            pltpu.VMEM((CHUNK, AW), jnp.float32),
            pltpu.VMEM_SHARED((NPAD, AW), jnp.float32),
            pltpu.SemaphoreType.DMA,
            pltpu.SemaphoreType.DMA,
            pltpu.SemaphoreType.DMA,
            pltpu.SemaphoreType.DMA,
        ],
    )
    def scatter_k(msg_hbm, idx_hbm3, zeros_hbm, out_hbm, idx_all,
                  rows0, rows1, acc_sh, ls0, ls1, ss0, ss1):
        cid = lax.axis_index("c")
        sid = lax.axis_index("s")
        wid = sid * 2 + cid

        rows_per_tile = NPAD // 16
        pltpu.sync_copy(zeros_hbm.at[pl.ds(sid * rows_per_tile, rows_per_tile)],
                        acc_sh.at[pl.ds(sid * rows_per_tile, rows_per_tile)])
        base = wid * n_edges_tile
        pltpu.sync_copy(idx_hbm3.at[wid], idx_all)

        plsc.subcore_barrier()
        rows = (rows0, rows1)
        lsem = (ls0, ls1)
        ssem = (ss0, ss1)

        def pair(p, carry):
            handles = []
            for b in range(2):
                j = p * 2 + b

                @pl.when(p >= 1)
                def _(b=b):
                    pltpu.make_async_copy(
                        rows[b], acc_sh.at[idx_all.at[0]], ssem[b]).wait()

                handles.append(pltpu.async_copy(
                    msg_hbm.at[pl.ds(base + j * CHUNK, CHUNK)], rows[b],
                    lsem[b]))
            for b in range(2):
                j = p * 2 + b
                handles[b].wait()
                pltpu.async_copy(rows[b], acc_sh.at[idx_all.at[j]],
                                 ssem[b], add=True)
            return carry

        lax.fori_loop(0, n_chunks // 2, pair, 0)
        for b in range(2):
            pltpu.make_async_copy(
                rows[b], acc_sh.at[idx_all.at[0]], ssem[b]).wait()
        plsc.subcore_barrier()

        @pl.when(sid == 0)
        def _():
            pltpu.sync_copy(acc_sh.at[pl.ds(0, N_NODES)], out_hbm.at[cid])

    return scatter_k


def _sc_scatter(msg, idx3, zeros):
    return _make_sc_scatter(idx3.shape[1])(msg, idx3, zeros)


def kernel(x, edge_index, edge_attr, batch, W_in, b_in, W_e1, b_e1, W_e2,
           b_e2, conv_bias, gru_W_ih, gru_W_hh, gru_b_ih, gru_b_hh,
           lstm_W_ih, lstm_W_hh, lstm_b_ih, lstm_b_hh, W_o1, b_o1, W_o2,
           b_o2):
    w_in_t = W_in.T
    b_in2 = b_in.reshape(1, S)
    w_e1_t = W_e1.T
    b_e12 = b_e1.reshape(1, S)
    gflat = W_e2.reshape(S, S, S).transpose(0, 2, 1).reshape(S * S, S)
    bmat = b_e2.reshape(S, S)
    eye = jnp.eye(S, dtype=jnp.float32)
    rep = jnp.kron(eye, jnp.ones((1, S), jnp.float32)).astype(jnp.bfloat16)
    tile = jnp.kron(jnp.ones((1, S), jnp.float32), eye).astype(jnp.bfloat16)
    gflat = gflat.astype(jnp.bfloat16)
    cb2 = conv_bias.reshape(1, S)
    wih_t = gru_W_ih.T
    whh_t = gru_W_hh.T
    bih2 = gru_b_ih.reshape(1, 3 * S)
    bhh2 = gru_b_hh.reshape(1, 3 * S)
    lwih_t = lstm_W_ih.T
    lwhh_t = lstm_W_hh.T
    lbih2 = lstm_b_ih.reshape(1, 4 * S)
    lbhh2 = lstm_b_hh.reshape(1, 4 * S)
    wo1_t = W_o1.T
    bo12 = b_o1.reshape(1, S)
    wo2_t = W_o2.T
    bo22 = b_o2.reshape(1, 1)

    pad_e = E_PAD - N_EDGES
    e_half = E_PAD // 2
    nch = N_CHUNKS // 2
    src_p = jnp.pad(edge_index[0], (0, pad_e))
    dst_p = jnp.pad(edge_index[1], (0, pad_e))
    src_h = [src_p[:e_half].reshape(NW, nch, CHUNK),
             src_p[e_half:].reshape(NW, nch, CHUNK)]
    dst_h = [dst_p[:e_half].reshape(NW, nch, CHUNK),
             dst_p[e_half:].reshape(NW, nch, CHUNK)]
    ea_pad = jnp.pad(edge_attr, ((0, pad_e), (0, 0)))
    batch2d = batch.reshape(N_NODES, 1)
    zeros_acc = jnp.zeros((NPAD, AW), jnp.float32)

    out = _in_linear(x, w_in_t, b_in2)
    eh = _eh(ea_pad, w_e1_t, b_e12)

    eh_h = [eh[:e_half], eh[e_half:]]
    h = out
    for _ in range(NUM_CONV):
        os0 = _sc_gather(out, src_h[0])
        os1 = _sc_gather(out, src_h[1])
        m0 = _msg(os0, eh_h[0], gflat, bmat, rep, tile, 0)
        m1 = _msg(os1, eh_h[1], gflat, bmat, rep, tile, e_half)
        p0 = _sc_scatter(m0, dst_h[0], zeros_acc)
        p1 = _sc_scatter(m1, dst_h[1], zeros_acc)
        h = _gru(p0, p1, h, wih_t, whh_t, bih2, bhh2, cb2)
        out = h

    y = _set2set(out, batch2d, lwih_t, lwhh_t, lbih2, lbhh2,
                 wo1_t, bo12, wo2_t, bo22)
    return y.reshape(-1)
